# baseline (speedup 1.0000x reference)
import functools

import jax
import jax.numpy as jnp
from jax import lax
from jax.experimental import pallas as pl
from jax.experimental.pallas import tpu as pltpu
from jax.experimental.pallas import tpu_sc as plsc

NC, NS = 2, 16
NW = NC * NS
CH = 128
NBUF = 4


@functools.lru_cache(maxsize=None)
def _make_gather(n_ch, d):
    mesh = plsc.VectorSubcoreMesh(
        core_axis_name="c", subcore_axis_name="s",
        num_cores=NC, num_subcores=NS,
    )

    @functools.partial(
        pl.kernel,
        out_type=jax.ShapeDtypeStruct((NW, n_ch, CH, d), jnp.float32),
        mesh=mesh,
        scratch_types=[
            pltpu.VMEM((n_ch, CH), jnp.int32),
            pltpu.VMEM((NBUF, CH, d), jnp.float32),
        ] + [pltpu.SemaphoreType.DMA] * NBUF,
        compiler_params=pltpu.CompilerParams(use_tc_tiling_on_sc=False),
    )
    def emb(ids_hbm, table_hbm, out_hbm, idx_v, rows_v, *sems):
        wid = lax.axis_index("s") * NC + lax.axis_index("c")
        pltpu.sync_copy(ids_hbm.at[wid], idx_v)

        def start(j, b):
            pltpu.async_copy(table_hbm.at[idx_v.at[j]], rows_v.at[b], sems[b])

        def wait(j, b):
            pltpu.make_async_copy(
                table_hbm.at[idx_v.at[j]], rows_v.at[b], sems[b]
            ).wait()

        for b in range(NBUF):
            start(b, b)

        @pl.loop(0, n_ch, step=NBUF)
        def _(g):
            for b in range(NBUF):
                j = g + b
                wait(j, b)
                pltpu.sync_copy(rows_v.at[b], out_hbm.at[wid, j])

                @pl.when(j + NBUF < n_ch)
                def _():
                    start(j + NBUF, b)

    return emb


def kernel(seq_ids, seq_len, table):
    batch, hist = seq_ids.shape
    d = table.shape[1]
    n_total = batch * hist
    n_ch = n_total // (NW * CH)
    ids = seq_ids.astype(jnp.int32).reshape(NW, n_ch, CH)
    out = _make_gather(n_ch, d)(ids, table)
    return out.reshape(batch, hist, d)
